# baseline (speedup 1.0000x reference)
import functools

import jax
import jax.numpy as jnp
from jax import lax
from jax.experimental import pallas as pl
from jax.experimental.pallas import tpu as pltpu
from jax.experimental.pallas import tpu_sc as plsc

NUM_NODES = 1000000
H_DIM = 64
N_LOOKUPS = 819200

NC, NS = 2, 16
NW = NC * NS
B_PER_W = N_LOOKUPS // NW
CHUNK = 128
N_CHUNKS = B_PER_W // CHUNK
NGROUPS = CHUNK // 16


@functools.partial(
    pl.kernel,
    out_type=jax.ShapeDtypeStruct((H_DIM, N_LOOKUPS), jnp.float32),
    mesh=plsc.VectorSubcoreMesh(core_axis_name="c", subcore_axis_name="s"),
    scratch_types=[
        pltpu.VMEM((B_PER_W,), jnp.int32),
        pltpu.VMEM((CHUNK, 128), jnp.float32),
        pltpu.VMEM((CHUNK, 128), jnp.float32),
        pltpu.VMEM((H_DIM, CHUNK), jnp.float32),
        pltpu.VMEM((H_DIM, CHUNK), jnp.float32),
        pltpu.VMEM((CHUNK,), jnp.int32),
        pltpu.VMEM((CHUNK,), jnp.int32),
        pltpu.VMEM((CHUNK,), jnp.int32),
        pltpu.VMEM((CHUNK,), jnp.int32),
        pltpu.SemaphoreType.DMA,
        pltpu.SemaphoreType.DMA,
        pltpu.SemaphoreType.DMA,
        pltpu.SemaphoreType.DMA,
    ],
    compiler_params=pltpu.CompilerParams(needs_layout_passes=False),
)
def _gather_kernel(idx_hbm, wp_hbm, ot_hbm, idx_v, g0, g1, o0, o1,
                   r0, r1, p0, p1, gs0, gs1, ss0, ss1):
    wid = lax.axis_index("s") * NC + lax.axis_index("c")
    base = wid * B_PER_W
    gbuf = (g0, g1)
    obuf = (o0, o1)
    rbuf = (r0, r1)
    pbuf = (p0, p1)
    gsem = (gs0, gs1)
    ssem = (ss0, ss1)

    pltpu.sync_copy(idx_hbm.at[pl.ds(base, B_PER_W)], idx_v)

    lane = lax.iota(jnp.int32, 16)
    rowv = [lane + (16 * jj) for jj in range(NGROUPS)]

    def prep(j, b):
        for k in range(NGROUPS):
            iv = idx_v[pl.ds(j * CHUNK + k * 16, 16)]
            rbuf[b][pl.ds(k * 16, 16)] = lax.shift_right_logical(iv, 1)
            pbuf[b][pl.ds(k * 16, 16)] = lax.shift_left(
                lax.bitwise_and(iv, 1), 6)

    def gather_start(b):
        pltpu.async_copy(wp_hbm.at[rbuf[b]], gbuf[b], gsem[b])

    def gather_wait(b):
        pltpu.make_async_copy(wp_hbm.at[pl.ds(0, CHUNK)], gbuf[b],
                              gsem[b]).wait()

    def store_start(j, b):
        pltpu.async_copy(obuf[b], ot_hbm.at[:, pl.ds(base + j * CHUNK, CHUNK)],
                         ssem[b])

    def store_wait(b):
        pltpu.make_async_copy(obuf[b], ot_hbm.at[:, pl.ds(0, CHUNK)],
                              ssem[b]).wait()

    def transpose_chunk(b):
        def cbody(c, carry):
            for jj in range(NGROUPS):
                bv = pbuf[b][pl.ds(jj * 16, 16)] + c
                vals = plsc.load_gather(gbuf[b], [rowv[jj], bv])
                obuf[b][c, pl.ds(jj * 16, 16)] = vals
            return carry
        lax.fori_loop(0, H_DIM, cbody, 0)

    prep(0, 0)
    gather_start(0)
    prep(1, 1)
    gather_start(1)

    def outer(jo, carry):
        for b in range(2):
            j = jo * 2 + b
            gather_wait(b)
            transpose_chunk(b)
            store_start(j, b)
            store_wait(b)
            prep(j + 2, b)
            gather_start(b)
        return carry

    lax.fori_loop(0, (N_CHUNKS - 2) // 2, outer, 0)

    for b in range(2):
        gather_wait(b)
        transpose_chunk(b)
        store_start(N_CHUNKS - 2 + b, b)
        store_wait(b)


def kernel(node_id, weight):
    node_id = jnp.squeeze(node_id).astype(jnp.int32)
    wp = weight.reshape(NUM_NODES // 2, 2 * H_DIM)
    ot = _gather_kernel(node_id, wp)
    return ot.T


# speedup vs baseline: 2.5677x; 2.5677x over previous
import functools

import jax
import jax.numpy as jnp
from jax import lax
from jax.experimental import pallas as pl
from jax.experimental.pallas import tpu as pltpu
from jax.experimental.pallas import tpu_sc as plsc

NUM_NODES = 1000000
H_DIM = 64
N_LOOKUPS = 819200

NC, NS = 2, 16
NW = NC * NS
B_PER_W = N_LOOKUPS // NW
CHUNK = 400
N_CHUNKS = B_PER_W // CHUNK


@functools.partial(
    pl.kernel,
    out_type=jax.ShapeDtypeStruct((N_LOOKUPS, 2 * H_DIM), jnp.float32),
    mesh=plsc.VectorSubcoreMesh(core_axis_name="c", subcore_axis_name="s"),
    scratch_types=[
        pltpu.VMEM((B_PER_W,), jnp.int32),
        pltpu.VMEM((CHUNK, 2 * H_DIM), jnp.float32),
        pltpu.VMEM((CHUNK, 2 * H_DIM), jnp.float32),
        pltpu.SemaphoreType.DMA,
        pltpu.SemaphoreType.DMA,
        pltpu.SemaphoreType.DMA,
        pltpu.SemaphoreType.DMA,
    ],
)
def _gather_kernel(idx_hbm, wp_hbm, out_hbm, idx_v, buf0, buf1,
                   g0, g1, s0, s1):
    wid = lax.axis_index("s") * NC + lax.axis_index("c")
    base = wid * B_PER_W
    bufs = (buf0, buf1)
    gsems = (g0, g1)
    ssems = (s0, s1)

    pltpu.sync_copy(idx_hbm.at[pl.ds(base, B_PER_W)], idx_v)

    def gather_start(i, b):
        pltpu.async_copy(wp_hbm.at[idx_v.at[pl.ds(i * CHUNK, CHUNK)]],
                         bufs[b], gsems[b])

    def gather_wait(b):
        pltpu.make_async_copy(wp_hbm.at[pl.ds(0, CHUNK)], bufs[b],
                              gsems[b]).wait()

    def store_start(i, b):
        pltpu.async_copy(bufs[b], out_hbm.at[pl.ds(base + i * CHUNK, CHUNK)],
                         ssems[b])

    def store_wait(b):
        pltpu.make_async_copy(bufs[b], out_hbm.at[pl.ds(base, CHUNK)],
                              ssems[b]).wait()

    gather_start(0, 0)
    gather_start(1, 1)

    def outer(jo, carry):
        for b in range(2):
            j = jo * 2 + b
            gather_wait(b)
            store_start(j, b)
            store_wait(b)
            gather_start(j + 2, b)
        return carry

    lax.fori_loop(0, (N_CHUNKS - 2) // 2, outer, 0)

    for b in range(2):
        gather_wait(b)
        store_start(N_CHUNKS - 2 + b, b)
        store_wait(b)


def kernel(node_id, weight):
    node_id = jnp.squeeze(node_id).astype(jnp.int32)
    wp = jnp.concatenate(
        [weight, jnp.zeros((NUM_NODES, H_DIM), jnp.float32)], axis=1)
    return _gather_kernel(node_id, wp)[:, :H_DIM]


# speedup vs baseline: 2.5689x; 1.0004x over previous
import functools

import jax
import jax.numpy as jnp
from jax import lax
from jax.experimental import pallas as pl
from jax.experimental.pallas import tpu as pltpu
from jax.experimental.pallas import tpu_sc as plsc

NUM_NODES = 1000000
H_DIM = 64
N_LOOKUPS = 819200

NC, NS = 2, 16
NW = NC * NS
B_PER_W = N_LOOKUPS // NW
CHUNK = 256
N_CHUNKS = B_PER_W // CHUNK
NBUF = 3


@functools.partial(
    pl.kernel,
    out_type=jax.ShapeDtypeStruct((N_LOOKUPS, 2 * H_DIM), jnp.float32),
    mesh=plsc.VectorSubcoreMesh(core_axis_name="c", subcore_axis_name="s"),
    scratch_types=[
        pltpu.VMEM((B_PER_W,), jnp.int32),
        pltpu.VMEM((CHUNK, 2 * H_DIM), jnp.float32),
        pltpu.VMEM((CHUNK, 2 * H_DIM), jnp.float32),
        pltpu.VMEM((CHUNK, 2 * H_DIM), jnp.float32),
        pltpu.SemaphoreType.DMA,
        pltpu.SemaphoreType.DMA,
        pltpu.SemaphoreType.DMA,
        pltpu.SemaphoreType.DMA,
        pltpu.SemaphoreType.DMA,
        pltpu.SemaphoreType.DMA,
    ],
)
def _gather_kernel(idx_hbm, wp_hbm, out_hbm, idx_v, buf0, buf1, buf2,
                   g0, g1, g2, s0, s1, s2):
    wid = lax.axis_index("s") * NC + lax.axis_index("c")
    base = wid * B_PER_W
    bufs = (buf0, buf1, buf2)
    gsems = (g0, g1, g2)
    ssems = (s0, s1, s2)

    pltpu.sync_copy(idx_hbm.at[pl.ds(base, B_PER_W)], idx_v)

    def gather_start(i, b):
        pltpu.async_copy(wp_hbm.at[idx_v.at[pl.ds(i * CHUNK, CHUNK)]],
                         bufs[b], gsems[b])

    def gather_wait(b):
        pltpu.make_async_copy(wp_hbm.at[pl.ds(0, CHUNK)], bufs[b],
                              gsems[b]).wait()

    def store_start(i, b):
        pltpu.async_copy(bufs[b], out_hbm.at[pl.ds(base + i * CHUNK, CHUNK)],
                         ssems[b])

    def store_wait(b):
        pltpu.make_async_copy(bufs[b], out_hbm.at[pl.ds(0, CHUNK)],
                              ssems[b]).wait()

    gather_start(0, 0)
    gather_start(1, 1)

    gather_wait(0)
    store_start(0, 0)
    gather_start(2, 2)

    def body(jo, carry):
        for u in range(3):
            j = jo * 3 + 1 + u
            b = (1 + u) % NBUF
            bn = (3 + u) % NBUF
            gather_wait(b)
            store_start(j, b)
            store_wait(bn)
            gather_start(j + 2, bn)
        return carry

    lax.fori_loop(0, (N_CHUNKS - 4) // 3, body, 0)

    store_wait((N_CHUNKS - 1) % NBUF)
    gather_start(N_CHUNKS - 1, (N_CHUNKS - 1) % NBUF)
    for j in range(N_CHUNKS - 3, N_CHUNKS):
        b = j % NBUF
        gather_wait(b)
        store_start(j, b)
    for j in range(N_CHUNKS - 3, N_CHUNKS):
        store_wait(j % NBUF)


def kernel(node_id, weight):
    node_id = jnp.squeeze(node_id).astype(jnp.int32)
    wp = jnp.concatenate(
        [weight, jnp.zeros((NUM_NODES, H_DIM), jnp.float32)], axis=1)
    return _gather_kernel(node_id, wp)[:, :H_DIM]
